# baseline (speedup 1.0000x reference)
import functools

import jax
import jax.numpy as jnp
from jax import lax
from jax.experimental import pallas as pl
from jax.experimental.pallas import tpu as pltpu
from jax.experimental.pallas import tpu_sc as plsc

_NC = 2
_NS = 16
_NW = _NC * _NS
_LANES = 128


def _sc_mesh():
    return plsc.VectorSubcoreMesh(
        core_axis_name="c", subcore_axis_name="s",
        num_cores=_NC, num_subcores=_NS)


def _make_deg_kernel(NP, RT):
    ZR = NP // _NS

    @functools.partial(
        pl.kernel,
        out_type=jax.ShapeDtypeStruct((_NC, NP, 16), jnp.float32),
        mesh=_sc_mesh(),
        scratch_types=[
            pltpu.VMEM((_LANES,), jnp.int32),
            pltpu.VMEM((_LANES,), jnp.int32),
            pltpu.VMEM((_LANES, 16), jnp.float32),
            pltpu.VMEM((_LANES, 16), jnp.float32),
            pltpu.VMEM_SHARED((NP, 16), jnp.float32),
        ],
    )
    def deg_k(sd_hbm, ones_hbm, zeros_hbm, out_hbm,
              sidx, didx, ones_s, ones_d, degsh):
        cid = lax.axis_index("c")
        sid = lax.axis_index("s")
        wid = cid * _NS + sid
        base = wid * RT
        pltpu.sync_copy(zeros_hbm, degsh.at[pl.ds(sid * ZR, ZR)])
        pltpu.sync_copy(ones_hbm.at[0], ones_s)
        pltpu.sync_copy(ones_hbm.at[1], ones_d)
        plsc.subcore_barrier()

        def body(s, carry):
            pltpu.sync_copy(sd_hbm.at[base + s, 0], sidx)
            pltpu.sync_copy(sd_hbm.at[base + s, 1], didx)
            pltpu.sync_copy(ones_s, degsh.at[sidx], add=True)
            pltpu.sync_copy(ones_d, degsh.at[didx], add=True)
            return carry

        lax.fori_loop(0, RT, body, 0)
        plsc.subcore_barrier()
        pltpu.sync_copy(degsh.at[pl.ds(sid * ZR, ZR)],
                        out_hbm.at[cid, pl.ds(sid * ZR, ZR)])

    return deg_k


def _make_agg_kernel(NP, RT0, RT1, D):
    ZR = NP // _NS

    @functools.partial(
        pl.kernel,
        out_type=jax.ShapeDtypeStruct((1, NP, D), jnp.float32),
        mesh=_sc_mesh(),
        scratch_types=[
            pltpu.VMEM((_LANES,), jnp.int32),
            pltpu.VMEM((_LANES,), jnp.int32),
            pltpu.VMEM((_LANES,), jnp.int32),
            pltpu.VMEM((_LANES,), jnp.int32),
            pltpu.VMEM((_LANES, D), jnp.float32),
            pltpu.VMEM((_LANES, D), jnp.float32),
            pltpu.VMEM_SHARED((NP, D), jnp.float32),
            pltpu.SemaphoreType.DMA,
            pltpu.SemaphoreType.DMA,
            pltpu.SemaphoreType.DMA,
            pltpu.SemaphoreType.DMA,
            pltpu.SemaphoreType.DMA,
            pltpu.SemaphoreType.DMA,
        ],
    )
    def agg_k(h_hbm, sd_hbm, out_hbm,
              sidx0, sidx1, didx0, didx1, rows0, rows1, aggsh,
              gsem0, gsem1, isem0, isem1, dsem0, dsem1):
        cid = lax.axis_index("c")
        sid = lax.axis_index("s")
        base = jnp.where(cid == 0, sid * RT0, _NS * RT0 + sid * RT1)
        rt = jnp.where(cid == 0, RT0, RT1)
        sidx = (sidx0, sidx1)
        didx = (didx0, didx1)
        rows = (rows0, rows1)
        gsem = (gsem0, gsem1)
        isem = (isem0, isem1)
        dsem = (dsem0, dsem1)

        zv = jnp.zeros((16,), jnp.float32)

        @pl.when(rt > 0)
        def _():
            def zbody(r, carry):
                for c in range(D // 16):
                    rows0[r, pl.ds(c * 16, 16)] = zv
                return carry

            lax.fori_loop(0, _LANES, zbody, 0)
            for z in range(ZR // _LANES):
                pltpu.sync_copy(
                    rows0, aggsh.at[pl.ds(sid * ZR + z * _LANES, _LANES)])
            pltpu.async_copy(sd_hbm.at[base, 0], sidx0, isem0)
            pltpu.async_copy(sd_hbm.at[base, 1], didx0, dsem0)
            pltpu.async_copy(sd_hbm.at[base + 1, 0], sidx1, isem1)
            pltpu.async_copy(sd_hbm.at[base + 1, 1], didx1, dsem1)

        plsc.subcore_barrier()

        @pl.when(rt > 0)
        def _():
            pltpu.make_async_copy(sd_hbm.at[base, 0], sidx0, isem0).wait()
            pltpu.async_copy(h_hbm.at[sidx0], rows0, gsem0)

        def body(g2, carry):
            for b in (0, 1):
                s = g2 * 2 + b
                nb = 1 - b

                @pl.when(s < rt - 1)
                def _():
                    pltpu.make_async_copy(sd_hbm.at[base + s + 1, 0],
                                          sidx[nb], isem[nb]).wait()
                    pltpu.async_copy(h_hbm.at[sidx[nb]], rows[nb], gsem[nb])

                pltpu.make_async_copy(h_hbm.at[sidx[b]],
                                      rows[b], gsem[b]).wait()
                pltpu.make_async_copy(sd_hbm.at[base + s, 1],
                                      didx[b], dsem[b]).wait()
                pltpu.sync_copy(rows[b], aggsh.at[didx[b]], add=True)

                @pl.when(s < rt - 2)
                def _():
                    pltpu.async_copy(sd_hbm.at[base + s + 2, 0],
                                     sidx[b], isem[b])
                    pltpu.async_copy(sd_hbm.at[base + s + 2, 1],
                                     didx[b], dsem[b])
            return carry

        lax.fori_loop(0, (rt + 1) // 2, body, 0)
        plsc.subcore_barrier()

        @pl.when(rt > 0)
        def _():
            pltpu.sync_copy(aggsh.at[pl.ds(sid * ZR, ZR)],
                            out_hbm.at[0, pl.ds(sid * ZR, ZR)])

    return agg_k


def _make_norm_kernel(NP, BR, D):
    def body(x_ref, dp_ref, xs_ref, ns_ref, nd_ref):
        dp = dp_ref[...]
        dsrc = dp[0, :, 0:1] + dp[1, :, 0:1]
        ddst = dp[0, :, 1:2] + dp[1, :, 1:2]
        ns = lax.rsqrt(jnp.maximum(dsrc, 1.0))
        nd = lax.rsqrt(jnp.maximum(ddst, 1.0))
        ns_ref[...] = ns
        nd_ref[...] = nd
        xs_ref[...] = x_ref[...] * ns

    return pl.pallas_call(
        body,
        grid=(NP // BR,),
        in_specs=[
            pl.BlockSpec((BR, D), lambda i: (i, 0)),
            pl.BlockSpec((2, BR, 16), lambda i: (0, i, 0)),
        ],
        out_specs=[
            pl.BlockSpec((BR, D), lambda i: (i, 0)),
            pl.BlockSpec((BR, 1), lambda i: (i, 0)),
            pl.BlockSpec((BR, 1), lambda i: (i, 0)),
        ],
        out_shape=[
            jax.ShapeDtypeStruct((NP, D), jnp.float32),
            jax.ShapeDtypeStruct((NP, 1), jnp.float32),
            jax.ShapeDtypeStruct((NP, 1), jnp.float32),
        ],
    )


def _make_mm_kernel(NP, BR, D, relu_and_src_scale):
    def body(ap_ref, w_ref, b_ref, nd_ref, ns_ref, out_ref):
        a = ap_ref[0]
        y = jnp.dot(a, w_ref[...], preferred_element_type=jnp.float32,
                    precision=lax.Precision.HIGHEST)
        y = y * nd_ref[...] + b_ref[...]
        if relu_and_src_scale:
            y = jnp.maximum(y, 0.0) * ns_ref[...]
        out_ref[...] = y

    return pl.pallas_call(
        body,
        grid=(NP // BR,),
        in_specs=[
            pl.BlockSpec((1, BR, D), lambda i: (0, i, 0)),
            pl.BlockSpec((D, D), lambda i: (0, 0)),
            pl.BlockSpec((1, D), lambda i: (0, 0)),
            pl.BlockSpec((BR, 1), lambda i: (i, 0)),
            pl.BlockSpec((BR, 1), lambda i: (i, 0)),
        ],
        out_specs=pl.BlockSpec((BR, D), lambda i: (i, 0)),
        out_shape=jax.ShapeDtypeStruct((NP, D), jnp.float32),
    )


def kernel(x, edge_index, W1, b1, W2, b2):
    N, D = x.shape
    E = edge_index.shape[1]
    BR = 512
    NP = -((N + 1) // -BR) * BR
    EP = -(E // -(2 * _NW * _LANES)) * (2 * _NW * _LANES)
    RT = EP // (_NW * _LANES)
    RT0 = 2 * RT
    RT1 = 2 * RT - RT0

    src = edge_index[0]
    dst = edge_index[1]
    pad = jnp.full((EP - E,), N, jnp.int32)
    src2d = jnp.concatenate([src, pad]).reshape(EP // _LANES, _LANES)
    dst2d = jnp.concatenate([dst, pad]).reshape(EP // _LANES, _LANES)
    sd = jnp.stack([src2d, dst2d], axis=1)
    x_pad = jnp.zeros((NP, D), jnp.float32).at[:N].set(x)

    eye2 = jnp.zeros((2, 16), jnp.float32).at[0, 0].set(1.0).at[1, 1].set(1.0)
    ones_pat = jnp.tile(eye2[:, None, :], (1, _LANES, 1))
    zeros2 = jnp.zeros((NP // _NS, 16), jnp.float32)

    deg_k = _make_deg_kernel(NP, RT)
    agg_k = _make_agg_kernel(NP, RT0, RT1, D)
    norm_k = _make_norm_kernel(NP, BR, D)
    mm_relu_k = _make_mm_kernel(NP, BR, D, True)
    mm_k = _make_mm_kernel(NP, BR, D, False)

    degp = deg_k(sd, ones_pat, zeros2)
    xs, nsrc, ndst = norm_k(x_pad, degp)
    agg1 = agg_k(xs, sd)
    h1s = mm_relu_k(agg1, W1, b1.reshape(1, D), ndst, nsrc)
    agg2 = agg_k(h1s, sd)
    out = mm_k(agg2, W2, b2.reshape(1, D), ndst, nsrc)
    return out[:N]


# speedup vs baseline: 1.1802x; 1.1802x over previous
import functools

import jax
import jax.numpy as jnp
from jax import lax
from jax.experimental import pallas as pl
from jax.experimental.pallas import tpu as pltpu
from jax.experimental.pallas import tpu_sc as plsc

_NC = 2
_NS = 16
_NW = _NC * _NS
_LANES = 128


def _sc_mesh():
    return plsc.VectorSubcoreMesh(
        core_axis_name="c", subcore_axis_name="s",
        num_cores=_NC, num_subcores=_NS)


def _make_deg_kernel(NP, RT):
    ZR = NP // _NS

    @functools.partial(
        pl.kernel,
        out_type=jax.ShapeDtypeStruct((_NC, NP, 16), jnp.float32),
        mesh=_sc_mesh(),
        scratch_types=[
            pltpu.VMEM((_LANES,), jnp.int32),
            pltpu.VMEM((_LANES,), jnp.int32),
            pltpu.VMEM((_LANES, 16), jnp.float32),
            pltpu.VMEM((_LANES, 16), jnp.float32),
            pltpu.VMEM_SHARED((NP, 16), jnp.float32),
        ],
    )
    def deg_k(sd_hbm, ones_hbm, zeros_hbm, out_hbm,
              sidx, didx, ones_s, ones_d, degsh):
        cid = lax.axis_index("c")
        sid = lax.axis_index("s")
        wid = cid * _NS + sid
        base = wid * RT
        pltpu.sync_copy(zeros_hbm, degsh.at[pl.ds(sid * ZR, ZR)])
        pltpu.sync_copy(ones_hbm.at[0], ones_s)
        pltpu.sync_copy(ones_hbm.at[1], ones_d)
        plsc.subcore_barrier()

        def body(s, carry):
            pltpu.sync_copy(sd_hbm.at[base + s, 0], sidx)
            pltpu.sync_copy(sd_hbm.at[base + s, 1], didx)
            pltpu.sync_copy(ones_s, degsh.at[sidx], add=True)
            pltpu.sync_copy(ones_d, degsh.at[didx], add=True)
            return carry

        lax.fori_loop(0, RT, body, 0)
        plsc.subcore_barrier()
        pltpu.sync_copy(degsh.at[pl.ds(sid * ZR, ZR)],
                        out_hbm.at[cid, pl.ds(sid * ZR, ZR)])

    return deg_k


def _make_agg_kernel(NP, RT0, RT1, D):
    ZR = NP // _NS

    @functools.partial(
        pl.kernel,
        out_type=jax.ShapeDtypeStruct((_NC, NP, D), jnp.float32),
        mesh=_sc_mesh(),
        scratch_types=[
            pltpu.VMEM((_LANES,), jnp.int32),
            pltpu.VMEM((_LANES,), jnp.int32),
            pltpu.VMEM((_LANES,), jnp.int32),
            pltpu.VMEM((_LANES,), jnp.int32),
            pltpu.VMEM((_LANES, D), jnp.float32),
            pltpu.VMEM((_LANES, D), jnp.float32),
            pltpu.VMEM_SHARED((NP, D), jnp.float32),
            pltpu.SemaphoreType.DMA,
            pltpu.SemaphoreType.DMA,
            pltpu.SemaphoreType.DMA,
            pltpu.SemaphoreType.DMA,
            pltpu.SemaphoreType.DMA,
            pltpu.SemaphoreType.DMA,
        ],
    )
    def agg_k(h_hbm, sd_hbm, out_hbm,
              sidx0, sidx1, didx0, didx1, rows0, rows1, aggsh,
              gsem0, gsem1, isem0, isem1, dsem0, dsem1):
        cid = lax.axis_index("c")
        sid = lax.axis_index("s")
        base = jnp.where(cid == 0, sid * RT0, _NS * RT0 + sid * RT1)
        rt = jnp.where(cid == 0, RT0, RT1)
        sidx = (sidx0, sidx1)
        didx = (didx0, didx1)
        rows = (rows0, rows1)
        gsem = (gsem0, gsem1)
        isem = (isem0, isem1)
        dsem = (dsem0, dsem1)

        zv = jnp.zeros((16,), jnp.float32)

        @pl.when(rt > 0)
        def _():
            def zbody(r, carry):
                for c in range(D // 16):
                    rows0[r, pl.ds(c * 16, 16)] = zv
                return carry

            lax.fori_loop(0, _LANES, zbody, 0)
            for z in range(ZR // _LANES):
                pltpu.sync_copy(
                    rows0, aggsh.at[pl.ds(sid * ZR + z * _LANES, _LANES)])
            pltpu.async_copy(sd_hbm.at[base, 0], sidx0, isem0)
            pltpu.async_copy(sd_hbm.at[base, 1], didx0, dsem0)
            pltpu.async_copy(sd_hbm.at[base + 1, 0], sidx1, isem1)
            pltpu.async_copy(sd_hbm.at[base + 1, 1], didx1, dsem1)

        plsc.subcore_barrier()

        @pl.when(rt > 0)
        def _():
            pltpu.make_async_copy(sd_hbm.at[base, 0], sidx0, isem0).wait()
            pltpu.async_copy(h_hbm.at[sidx0], rows0, gsem0)

        def body(g2, carry):
            for b in (0, 1):
                s = g2 * 2 + b
                nb = 1 - b

                @pl.when(s < rt - 1)
                def _():
                    pltpu.make_async_copy(sd_hbm.at[base + s + 1, 0],
                                          sidx[nb], isem[nb]).wait()
                    pltpu.async_copy(h_hbm.at[sidx[nb]], rows[nb], gsem[nb])

                pltpu.make_async_copy(h_hbm.at[sidx[b]],
                                      rows[b], gsem[b]).wait()
                pltpu.make_async_copy(sd_hbm.at[base + s, 1],
                                      didx[b], dsem[b]).wait()
                pltpu.sync_copy(rows[b], aggsh.at[didx[b]], add=True)

                @pl.when(s < rt - 2)
                def _():
                    pltpu.async_copy(sd_hbm.at[base + s + 2, 0],
                                     sidx[b], isem[b])
                    pltpu.async_copy(sd_hbm.at[base + s + 2, 1],
                                     didx[b], dsem[b])
            return carry

        lax.fori_loop(0, (rt + 1) // 2, body, 0)
        plsc.subcore_barrier()

        @pl.when(rt > 0)
        def _():
            pltpu.sync_copy(aggsh.at[pl.ds(sid * ZR, ZR)],
                            out_hbm.at[cid, pl.ds(sid * ZR, ZR)])

    return agg_k


def _make_norm_kernel(NP, BR, D):
    def body(x_ref, dp_ref, xs_ref, ns_ref, nd_ref):
        dp = dp_ref[...]
        dsrc = dp[0, :, 0:1] + dp[1, :, 0:1]
        ddst = dp[0, :, 1:2] + dp[1, :, 1:2]
        ns = lax.rsqrt(jnp.maximum(dsrc, 1.0))
        nd = lax.rsqrt(jnp.maximum(ddst, 1.0))
        ns_ref[...] = ns
        nd_ref[...] = nd
        xs_ref[...] = x_ref[...] * ns

    return pl.pallas_call(
        body,
        grid=(NP // BR,),
        in_specs=[
            pl.BlockSpec((BR, D), lambda i: (i, 0)),
            pl.BlockSpec((2, BR, 16), lambda i: (0, i, 0)),
        ],
        out_specs=[
            pl.BlockSpec((BR, D), lambda i: (i, 0)),
            pl.BlockSpec((BR, 1), lambda i: (i, 0)),
            pl.BlockSpec((BR, 1), lambda i: (i, 0)),
        ],
        out_shape=[
            jax.ShapeDtypeStruct((NP, D), jnp.float32),
            jax.ShapeDtypeStruct((NP, 1), jnp.float32),
            jax.ShapeDtypeStruct((NP, 1), jnp.float32),
        ],
    )


def _make_mm_kernel(NP, BR, D, relu_and_src_scale):
    def body(ap_ref, w_ref, b_ref, nd_ref, ns_ref, out_ref):
        a = ap_ref[0] + ap_ref[1]
        y = jnp.dot(a, w_ref[...], preferred_element_type=jnp.float32,
                    precision=lax.Precision.HIGHEST)
        y = y * nd_ref[...] + b_ref[...]
        if relu_and_src_scale:
            y = jnp.maximum(y, 0.0) * ns_ref[...]
        out_ref[...] = y

    return pl.pallas_call(
        body,
        grid=(NP // BR,),
        in_specs=[
            pl.BlockSpec((2, BR, D), lambda i: (0, i, 0)),
            pl.BlockSpec((D, D), lambda i: (0, 0)),
            pl.BlockSpec((1, D), lambda i: (0, 0)),
            pl.BlockSpec((BR, 1), lambda i: (i, 0)),
            pl.BlockSpec((BR, 1), lambda i: (i, 0)),
        ],
        out_specs=pl.BlockSpec((BR, D), lambda i: (i, 0)),
        out_shape=jax.ShapeDtypeStruct((NP, D), jnp.float32),
    )


def kernel(x, edge_index, W1, b1, W2, b2):
    N, D = x.shape
    E = edge_index.shape[1]
    BR = 512
    NP = -((N + 1) // -BR) * BR
    EP = -(E // -(2 * _NW * _LANES)) * (2 * _NW * _LANES)
    RT = EP // (_NW * _LANES)
    RT0 = (2 * RT * 3) // 4
    RT1 = 2 * RT - RT0

    src = edge_index[0]
    dst = edge_index[1]
    pad = jnp.full((EP - E,), N, jnp.int32)
    src2d = jnp.concatenate([src, pad]).reshape(EP // _LANES, _LANES)
    dst2d = jnp.concatenate([dst, pad]).reshape(EP // _LANES, _LANES)
    sd = jnp.stack([src2d, dst2d], axis=1)
    x_pad = jnp.zeros((NP, D), jnp.float32).at[:N].set(x)

    eye2 = jnp.zeros((2, 16), jnp.float32).at[0, 0].set(1.0).at[1, 1].set(1.0)
    ones_pat = jnp.tile(eye2[:, None, :], (1, _LANES, 1))
    zeros2 = jnp.zeros((NP // _NS, 16), jnp.float32)

    deg_k = _make_deg_kernel(NP, RT)
    agg_k = _make_agg_kernel(NP, RT0, RT1, D)
    norm_k = _make_norm_kernel(NP, BR, D)
    mm_relu_k = _make_mm_kernel(NP, BR, D, True)
    mm_k = _make_mm_kernel(NP, BR, D, False)

    degp = deg_k(sd, ones_pat, zeros2)
    xs, nsrc, ndst = norm_k(x_pad, degp)
    agg1 = agg_k(xs, sd)
    h1s = mm_relu_k(agg1, W1, b1.reshape(1, D), ndst, nsrc)
    agg2 = agg_k(h1s, sd)
    out = mm_k(agg2, W2, b2.reshape(1, D), ndst, nsrc)
    return out[:N]


# speedup vs baseline: 1.2596x; 1.0673x over previous
import functools

import jax
import jax.numpy as jnp
from jax import lax
from jax.experimental import pallas as pl
from jax.experimental.pallas import tpu as pltpu
from jax.experimental.pallas import tpu_sc as plsc

_NC = 2
_NS = 16
_NW = _NC * _NS
_LANES = 128


def _sc_mesh():
    return plsc.VectorSubcoreMesh(
        core_axis_name="c", subcore_axis_name="s",
        num_cores=_NC, num_subcores=_NS)


def _make_deg_kernel(NP, RT):
    ZR = NP // _NS

    @functools.partial(
        pl.kernel,
        out_type=jax.ShapeDtypeStruct((_NC, NP, 16), jnp.float32),
        mesh=_sc_mesh(),
        scratch_types=[
            pltpu.VMEM((_LANES,), jnp.int32),
            pltpu.VMEM((_LANES,), jnp.int32),
            pltpu.VMEM((_LANES,), jnp.int32),
            pltpu.VMEM((_LANES,), jnp.int32),
            pltpu.VMEM((_LANES, 16), jnp.float32),
            pltpu.VMEM((_LANES, 16), jnp.float32),
            pltpu.VMEM_SHARED((NP, 16), jnp.float32),
            pltpu.SemaphoreType.DMA,
            pltpu.SemaphoreType.DMA,
            pltpu.SemaphoreType.DMA,
            pltpu.SemaphoreType.DMA,
        ],
    )
    def deg_k(sd_hbm, ones_hbm, zeros_hbm, out_hbm,
              sidx0, sidx1, didx0, didx1, ones_s, ones_d, degsh,
              isem0, isem1, dsem0, dsem1):
        cid = lax.axis_index("c")
        sid = lax.axis_index("s")
        wid = cid * _NS + sid
        base = wid * RT
        sidx = (sidx0, sidx1)
        didx = (didx0, didx1)
        isem = (isem0, isem1)
        dsem = (dsem0, dsem1)
        pltpu.sync_copy(zeros_hbm, degsh.at[pl.ds(sid * ZR, ZR)])
        pltpu.sync_copy(ones_hbm.at[0], ones_s)
        pltpu.sync_copy(ones_hbm.at[1], ones_d)
        pltpu.async_copy(sd_hbm.at[base, 0], sidx0, isem0)
        pltpu.async_copy(sd_hbm.at[base, 1], didx0, dsem0)
        pltpu.async_copy(sd_hbm.at[base + 1, 0], sidx1, isem1)
        pltpu.async_copy(sd_hbm.at[base + 1, 1], didx1, dsem1)
        plsc.subcore_barrier()

        def body(g2, carry):
            for b in (0, 1):
                s = g2 * 2 + b
                pltpu.make_async_copy(sd_hbm.at[base + s, 0],
                                      sidx[b], isem[b]).wait()
                pltpu.make_async_copy(sd_hbm.at[base + s, 1],
                                      didx[b], dsem[b]).wait()
                pltpu.sync_copy(ones_s, degsh.at[sidx[b]], add=True)
                pltpu.sync_copy(ones_d, degsh.at[didx[b]], add=True)

                @pl.when(s < RT - 2)
                def _():
                    pltpu.async_copy(sd_hbm.at[base + s + 2, 0],
                                     sidx[b], isem[b])
                    pltpu.async_copy(sd_hbm.at[base + s + 2, 1],
                                     didx[b], dsem[b])
            return carry

        lax.fori_loop(0, RT // 2, body, 0)
        plsc.subcore_barrier()
        pltpu.sync_copy(degsh.at[pl.ds(sid * ZR, ZR)],
                        out_hbm.at[cid, pl.ds(sid * ZR, ZR)])

    return deg_k


def _make_agg_kernel(NP, RT0, RT1, D):
    ZR = NP // _NS

    @functools.partial(
        pl.kernel,
        out_type=jax.ShapeDtypeStruct((_NC, NP, D), jnp.float32),
        mesh=_sc_mesh(),
        scratch_types=[
            pltpu.VMEM((_LANES,), jnp.int32),
            pltpu.VMEM((_LANES,), jnp.int32),
            pltpu.VMEM((_LANES,), jnp.int32),
            pltpu.VMEM((_LANES,), jnp.int32),
            pltpu.VMEM((_LANES, D), jnp.float32),
            pltpu.VMEM((_LANES, D), jnp.float32),
            pltpu.VMEM_SHARED((NP, D), jnp.float32),
            pltpu.SemaphoreType.DMA,
            pltpu.SemaphoreType.DMA,
            pltpu.SemaphoreType.DMA,
            pltpu.SemaphoreType.DMA,
            pltpu.SemaphoreType.DMA,
            pltpu.SemaphoreType.DMA,
        ],
    )
    def agg_k(h_hbm, sd_hbm, out_hbm,
              sidx0, sidx1, didx0, didx1, rows0, rows1, aggsh,
              gsem0, gsem1, isem0, isem1, dsem0, dsem1):
        cid = lax.axis_index("c")
        sid = lax.axis_index("s")
        base = jnp.where(cid == 0, sid * RT0, _NS * RT0 + sid * RT1)
        rt = jnp.where(cid == 0, RT0, RT1)
        sidx = (sidx0, sidx1)
        didx = (didx0, didx1)
        rows = (rows0, rows1)
        gsem = (gsem0, gsem1)
        isem = (isem0, isem1)
        dsem = (dsem0, dsem1)

        zv = jnp.zeros((16,), jnp.float32)

        @pl.when(rt > 0)
        def _():
            def zbody(r, carry):
                for c in range(D // 16):
                    rows0[r, pl.ds(c * 16, 16)] = zv
                return carry

            lax.fori_loop(0, _LANES, zbody, 0)
            for z in range(ZR // _LANES):
                pltpu.sync_copy(
                    rows0, aggsh.at[pl.ds(sid * ZR + z * _LANES, _LANES)])
            pltpu.async_copy(sd_hbm.at[base, 0], sidx0, isem0)
            pltpu.async_copy(sd_hbm.at[base, 1], didx0, dsem0)
            pltpu.async_copy(sd_hbm.at[base + 1, 0], sidx1, isem1)
            pltpu.async_copy(sd_hbm.at[base + 1, 1], didx1, dsem1)

        plsc.subcore_barrier()

        @pl.when(rt > 0)
        def _():
            pltpu.make_async_copy(sd_hbm.at[base, 0], sidx0, isem0).wait()
            pltpu.async_copy(h_hbm.at[sidx0], rows0, gsem0)

        def body(g2, carry):
            for b in (0, 1):
                s = g2 * 2 + b
                nb = 1 - b

                @pl.when(s < rt - 1)
                def _():
                    pltpu.make_async_copy(sd_hbm.at[base + s + 1, 0],
                                          sidx[nb], isem[nb]).wait()
                    pltpu.async_copy(h_hbm.at[sidx[nb]], rows[nb], gsem[nb])

                pltpu.make_async_copy(h_hbm.at[sidx[b]],
                                      rows[b], gsem[b]).wait()
                pltpu.make_async_copy(sd_hbm.at[base + s, 1],
                                      didx[b], dsem[b]).wait()
                pltpu.sync_copy(rows[b], aggsh.at[didx[b]], add=True)

                @pl.when(s < rt - 2)
                def _():
                    pltpu.async_copy(sd_hbm.at[base + s + 2, 0],
                                     sidx[b], isem[b])
                    pltpu.async_copy(sd_hbm.at[base + s + 2, 1],
                                     didx[b], dsem[b])
            return carry

        lax.fori_loop(0, (rt + 1) // 2, body, 0)
        plsc.subcore_barrier()

        @pl.when(rt > 0)
        def _():
            pltpu.sync_copy(aggsh.at[pl.ds(sid * ZR, ZR)],
                            out_hbm.at[cid, pl.ds(sid * ZR, ZR)])

    return agg_k


def _make_norm_kernel(NP, BR, D):
    def body(x_ref, dp_ref, xs_ref, ns_ref, nd_ref):
        dp = dp_ref[...]
        dsrc = dp[0, :, 0:1] + dp[1, :, 0:1]
        ddst = dp[0, :, 1:2] + dp[1, :, 1:2]
        ns = lax.rsqrt(jnp.maximum(dsrc, 1.0))
        nd = lax.rsqrt(jnp.maximum(ddst, 1.0))
        ns_ref[...] = ns
        nd_ref[...] = nd
        xs_ref[...] = x_ref[...] * ns

    return pl.pallas_call(
        body,
        grid=(NP // BR,),
        in_specs=[
            pl.BlockSpec((BR, D), lambda i: (i, 0)),
            pl.BlockSpec((2, BR, 16), lambda i: (0, i, 0)),
        ],
        out_specs=[
            pl.BlockSpec((BR, D), lambda i: (i, 0)),
            pl.BlockSpec((BR, 1), lambda i: (i, 0)),
            pl.BlockSpec((BR, 1), lambda i: (i, 0)),
        ],
        out_shape=[
            jax.ShapeDtypeStruct((NP, D), jnp.float32),
            jax.ShapeDtypeStruct((NP, 1), jnp.float32),
            jax.ShapeDtypeStruct((NP, 1), jnp.float32),
        ],
    )


def _make_mm_kernel(NP, BR, D, relu_and_src_scale):
    def body(ap_ref, w_ref, b_ref, nd_ref, ns_ref, out_ref):
        a = ap_ref[0] + ap_ref[1]
        y = jnp.dot(a, w_ref[...], preferred_element_type=jnp.float32,
                    precision=lax.Precision.HIGHEST)
        y = y * nd_ref[...] + b_ref[...]
        if relu_and_src_scale:
            y = jnp.maximum(y, 0.0) * ns_ref[...]
        out_ref[...] = y

    return pl.pallas_call(
        body,
        grid=(NP // BR,),
        in_specs=[
            pl.BlockSpec((2, BR, D), lambda i: (0, i, 0)),
            pl.BlockSpec((D, D), lambda i: (0, 0)),
            pl.BlockSpec((1, D), lambda i: (0, 0)),
            pl.BlockSpec((BR, 1), lambda i: (i, 0)),
            pl.BlockSpec((BR, 1), lambda i: (i, 0)),
        ],
        out_specs=pl.BlockSpec((BR, D), lambda i: (i, 0)),
        out_shape=jax.ShapeDtypeStruct((NP, D), jnp.float32),
    )


def kernel(x, edge_index, W1, b1, W2, b2):
    N, D = x.shape
    E = edge_index.shape[1]
    BR = 512
    NP = -((N + 1) // -BR) * BR
    EP = -(E // -(2 * _NW * _LANES)) * (2 * _NW * _LANES)
    RT = EP // (_NW * _LANES)
    RT0 = (2 * RT * 3) // 4
    RT1 = 2 * RT - RT0

    src = edge_index[0]
    dst = edge_index[1]
    pad = jnp.full((EP - E,), N, jnp.int32)
    src2d = jnp.concatenate([src, pad]).reshape(EP // _LANES, _LANES)
    dst2d = jnp.concatenate([dst, pad]).reshape(EP // _LANES, _LANES)
    sd = jnp.stack([src2d, dst2d], axis=1)
    x_pad = jnp.zeros((NP, D), jnp.float32).at[:N].set(x)

    eye2 = jnp.zeros((2, 16), jnp.float32).at[0, 0].set(1.0).at[1, 1].set(1.0)
    ones_pat = jnp.tile(eye2[:, None, :], (1, _LANES, 1))
    zeros2 = jnp.zeros((NP // _NS, 16), jnp.float32)

    deg_k = _make_deg_kernel(NP, RT)
    agg_k = _make_agg_kernel(NP, RT0, RT1, D)
    norm_k = _make_norm_kernel(NP, BR, D)
    mm_relu_k = _make_mm_kernel(NP, BR, D, True)
    mm_k = _make_mm_kernel(NP, BR, D, False)

    degp = deg_k(sd, ones_pat, zeros2)
    xs, nsrc, ndst = norm_k(x_pad, degp)
    agg1 = agg_k(xs, sd)
    h1s = mm_relu_k(agg1, W1, b1.reshape(1, D), ndst, nsrc)
    agg2 = agg_k(h1s, sd)
    out = mm_k(agg2, W2, b2.reshape(1, D), ndst, nsrc)
    return out[:N]
